# baseline (speedup 1.0000x reference)
import jax
import jax.numpy as jnp
from jax import lax
from jax.experimental import pallas as pl
from jax.experimental.pallas import tpu as pltpu, tpu_sc as plsc

_ALPHA = 0.7
_NSTATES = 128
_B, _S, _V = 4, 2048, 1024
_N = _B * _S
_CHUNK = _N // 32
_ROWS = 1024
_PACKED_WORDS = _NSTATES * _V // 4

_i32 = jnp.int32
_f32 = jnp.float32


def _iota16():
    return lax.iota(_i32, 16)


def _prep_body(trans_hbm, st_hbm, rejt_hbm, packed_hbm, isrej_hbm,
               tcol, stv, isrejv, rejtv, packedv, sem):
    c = lax.axis_index("c")
    s = lax.axis_index("s")
    wid = c * 16 + s
    v0 = wid * 32
    blk = lax.div(wid, 4)
    sub = jnp.bitwise_and(wid, 3)

    cp1 = pltpu.async_copy(st_hbm, stv, sem)
    cp2 = pltpu.async_copy(trans_hbm.at[:, pl.ds(blk * 128, 128)], tcol, sem)
    cp1.wait()
    cp2.wait()

    for k in range(8):
        sv = stv[pl.ds(k * 16, 16)]
        isrejv[pl.ds(k * 16, 16)] = jnp.where(sv == -1, 1.0, 0.0).astype(_f32)

    @pl.when(wid == 0)
    def _():
        pltpu.sync_copy(isrejv, isrej_hbm)

    for vl in range(32):
        colv = jnp.full((16,), sub * 32 + vl, _i32)
        for k in range(8):
            tv = plsc.load_gather(tcol, [_iota16() + 16 * k, colv])
            rj = plsc.load_gather(isrejv, [tv])
            rejtv[vl, pl.ds(16 * k, 16)] = rj
        for half in range(2):
            row4 = lax.shift_left(_iota16() + 16 * half, 2)
            g0 = plsc.load_gather(tcol, [row4, colv])
            g1 = plsc.load_gather(tcol, [row4 + 1, colv])
            g2 = plsc.load_gather(tcol, [row4 + 2, colv])
            g3 = plsc.load_gather(tcol, [row4 + 3, colv])
            word = jnp.bitwise_or(
                jnp.bitwise_or(g0, lax.shift_left(g1, 8)),
                jnp.bitwise_or(lax.shift_left(g2, 16), lax.shift_left(g3, 24)))
            packedv[pl.ds(32 * vl + 16 * half, 16)] = word
    cp3 = pltpu.async_copy(rejtv, rejt_hbm.at[pl.ds(v0, 32), :], sem)
    cp4 = pltpu.async_copy(packedv, packed_hbm.at[pl.ds(wid * 1024, 1024)], sem)
    cp3.wait()
    cp4.wait()


def _run_prep(transition, state_types):
    mesh = plsc.VectorSubcoreMesh(core_axis_name="c", subcore_axis_name="s")
    return pl.kernel(
        _prep_body,
        out_type=(
            jax.ShapeDtypeStruct((_V, _NSTATES), _f32),
            jax.ShapeDtypeStruct((_PACKED_WORDS,), _i32),
            jax.ShapeDtypeStruct((_NSTATES,), _f32),
        ),
        mesh=mesh,
        compiler_params=pltpu.CompilerParams(needs_layout_passes=False),
        scratch_types=(
            pltpu.VMEM((_NSTATES, 128), _i32),
            pltpu.VMEM((_NSTATES,), _i32),
            pltpu.VMEM((_NSTATES,), _f32),
            pltpu.VMEM((32, _NSTATES), _f32),
            pltpu.VMEM((1024,), _i32),
            pltpu.SemaphoreType.DMA,
        ),
    )(transition, state_types)


def _argmax_body(inp_ref, tok_ref):
    iota3 = lax.broadcasted_iota(_i32, (8, 128, _V), 2)
    x3 = inp_ref[...].reshape(8, 128, _V)
    xm = jnp.max(x3, axis=2, keepdims=True)
    tok3 = jnp.min(jnp.where(x3 == xm, iota3, _V), axis=2, keepdims=True)
    tok_ref[...] = tok3.reshape(8, 128)


def _run_argmax(inp2):
    return pl.pallas_call(
        _argmax_body,
        grid=(_N // _ROWS,),
        compiler_params=pltpu.CompilerParams(
            dimension_semantics=("arbitrary",)),
        in_specs=[pl.BlockSpec((_ROWS, _V), lambda i: (i, 0))],
        out_specs=[pl.BlockSpec((8, 128), lambda i: (i, 0))],
        out_shape=[jax.ShapeDtypeStruct((_N // 128, 128), _i32)],
    )(inp2)


def _soft_body(pred_ref, tgt_ref, rejt_ref, ce_ref, r_ref):
    p3 = pred_ref[...].reshape(8, 128, _V)
    iota3 = lax.broadcasted_iota(_i32, (8, 128, _V), 2)
    m = jnp.max(p3, axis=2, keepdims=True)
    e3 = jnp.exp(p3 - m)
    z = jnp.sum(e3, axis=2, keepdims=True)
    tgt3 = tgt_ref[...][..., None]
    pt = jnp.sum(jnp.where(iota3 == tgt3, p3, 0.0), axis=2, keepdims=True)
    ce_ref[...] = (m + jnp.log(z) - pt).reshape(8, 128)

    pn = (e3 / z).reshape(_ROWS, _V)
    r_ref[...] = jnp.dot(pn, rejt_ref[...], preferred_element_type=_f32)


def _run_soft(pred2, tgt64, rejt):
    return pl.pallas_call(
        _soft_body,
        grid=(_N // _ROWS,),
        compiler_params=pltpu.CompilerParams(
            dimension_semantics=("arbitrary",)),
        in_specs=[
            pl.BlockSpec((_ROWS, _V), lambda i: (i, 0)),
            pl.BlockSpec((8, 128), lambda i: (i, 0)),
            pl.BlockSpec((_V, _NSTATES), lambda i: (0, 0)),
        ],
        out_specs=[
            pl.BlockSpec((8, 128), lambda i: (i, 0)),
            pl.BlockSpec((_ROWS, _NSTATES), lambda i: (i, 0)),
        ],
        out_shape=[
            jax.ShapeDtypeStruct((_N // 128, 128), _f32),
            jax.ShapeDtypeStruct((_N, _NSTATES), _f32),
        ],
    )(pred2, tgt64, rejt)


def _scan_body(tok_hbm, tgt_hbm, packed_hbm, st_hbm,
               states_hbm, w_hbm, fstage_hbm,
               packedv, stored, tokv, tgtv, stv, isrejv,
               fbuf, frow, statev, wv, tokw8, tgtw8, sem):
    c = lax.axis_index("c")
    s = lax.axis_index("s")
    wid = c * 16 + s
    chunk = jnp.bitwise_and(s, 7)
    win0 = 8 * lax.div(wid, 4)
    m4 = jnp.bitwise_and(wid, 3)
    copies = [
        pltpu.async_copy(packed_hbm, packedv, sem),
        pltpu.async_copy(tok_hbm.at[pl.ds(win0, 8), :], tokw8, sem),
        pltpu.async_copy(tgt_hbm.at[pl.ds(win0, 8), :], tgtw8, sem),
        pltpu.async_copy(st_hbm, stv, sem),
    ]
    for cp in copies:
        cp.wait()
    for i in range(16):
        row = 2 * m4 + i // 8
        col = (16 * i) % 128
        tokv[pl.ds(16 * i, 16)] = tokw8[row, pl.ds(col, 16)]
        tgtv[pl.ds(16 * i, 16)] = tgtw8[row, pl.ds(col, 16)]
    for k in range(8):
        sv = stv[pl.ds(k * 16, 16)]
        isrejv[pl.ds(k * 16, 16)] = jnp.where(sv == -1, 1.0, 0.0).astype(_f32)

    def step(p, carry):
        pvec = carry[0]
        fs = list(carry[1:])
        new = [pvec + 1]
        for sub in range(4):
            tokbc = plsc.load_gather(tokv, [pvec + 64 * sub])
            tokw = lax.shift_left(tokbc, 5)
            for k in range(8):
                f = fs[sub * 8 + k]
                idx = tokw + lax.shift_right_logical(f, 2)
                g = plsc.load_gather(packedv, [idx])
                fsh = lax.shift_left(jnp.bitwise_and(f, 3), 3)
                nxt = jnp.bitwise_and(lax.shift_right_logical(g, fsh), 255)
                stored[64 * sub + p, pl.ds(16 * k, 16)] = nxt
                new.append(nxt)
        return tuple(new)

    finit = (jnp.zeros((16,), _i32),) + tuple(
        _iota16() + 16 * k for k in range(8)) * 4
    lax.fori_loop(0, 64, step, finit)

    fk = [stored[63, pl.ds(16 * k, 16)] for k in range(8)]
    for level in range(1, 4):
        rowv = jnp.full((16,), 64 * level + 63, _i32)
        fk = [plsc.load_gather(stored, [rowv, f]) for f in fk]
    for k in range(8):
        frow[pl.ds(16 * k, 16)] = fk[k]
    pltpu.sync_copy(frow, fstage_hbm.at[pl.ds(wid * _NSTATES, _NSTATES)])
    plsc.subcore_barrier()

    rowbase = c * 16 + jnp.bitwise_and(s, 8)
    pltpu.sync_copy(
        fstage_hbm.at[pl.ds(rowbase * _NSTATES, 8 * _NSTATES)], fbuf)
    s0 = jnp.asarray(0, _i32)
    for k in range(7):
        val = jnp.max(
            plsc.load_gather(fbuf, [jnp.full((16,), k * _NSTATES + s0, _i32)]))
        s0 = jnp.where(chunk > k, val, s0)

    s_sub = [s0]
    for t in range(1, 4):
        prev = s_sub[t - 1]
        v = jnp.max(plsc.load_gather(
            stored, [jnp.full((16,), 64 * t - 1, _i32),
                     jnp.full((16,), prev, _i32)]))
        s_sub.append(v)
    s_subv = [jnp.full((16,), x, _i32) for x in s_sub]

    for i in range(16):
        sub = i // 4
        s0vec = s_subv[sub]
        plv = _iota16() + 16 * i
        pm1 = jnp.maximum(plv - 1, 0)
        prev = plsc.load_gather(stored, [pm1, s0vec])
        states = jnp.where(plv == 64 * sub, s0vec, prev)
        tgt = tgtv[pl.ds(16 * i, 16)]
        idx = lax.shift_left(tgt, 5) + lax.shift_right_logical(states, 2)
        g = plsc.load_gather(packedv, [idx])
        sh = lax.shift_left(jnp.bitwise_and(states, 3), 3)
        nxt = jnp.bitwise_and(lax.shift_right_logical(g, sh), 255)
        rj = plsc.load_gather(isrejv, [nxt])
        statev[pl.ds(16 * i, 16)] = states
        wv[pl.ds(16 * i, 16)] = 1.0 - 0.95 * rj

    g0 = wid * _CHUNK
    cp5 = pltpu.async_copy(statev, states_hbm.at[pl.ds(g0, _CHUNK)], sem)
    cp6 = pltpu.async_copy(wv, w_hbm.at[pl.ds(g0, _CHUNK)], sem)
    cp5.wait()
    cp6.wait()


def _run_scan(tokens, targets, packed, state_types):
    mesh = plsc.VectorSubcoreMesh(core_axis_name="c", subcore_axis_name="s")
    return pl.kernel(
        _scan_body,
        out_type=(
            jax.ShapeDtypeStruct((_N,), _i32),
            jax.ShapeDtypeStruct((_N,), _f32),
            jax.ShapeDtypeStruct((32 * _NSTATES,), _i32),
        ),
        mesh=mesh,
        compiler_params=pltpu.CompilerParams(needs_layout_passes=False),
        scratch_types=(
            pltpu.VMEM((_PACKED_WORDS,), _i32),
            pltpu.VMEM((_CHUNK, _NSTATES), _i32),
            pltpu.VMEM((_CHUNK,), _i32),
            pltpu.VMEM((_CHUNK,), _i32),
            pltpu.VMEM((_NSTATES,), _i32),
            pltpu.VMEM((_NSTATES,), _f32),
            pltpu.VMEM((8 * _NSTATES,), _i32),
            pltpu.VMEM((_NSTATES,), _i32),
            pltpu.VMEM((_CHUNK,), _i32),
            pltpu.VMEM((_CHUNK,), _f32),
            pltpu.VMEM((8, 128), _i32),
            pltpu.VMEM((8, 128), _i32),
            pltpu.SemaphoreType.DMA,
        ),
    )(tokens, targets, packed, state_types)


def _select_body(states_hbm, w_hbm, ce_hbm, r_hbm,
                 partials_hbm, out_hbm,
                 rv, statev, wv, cev, cew8, pvec, redbuf, sem):
    c = lax.axis_index("c")
    s = lax.axis_index("s")
    wid = c * 16 + s
    win0 = 8 * lax.div(wid, 4)
    m4 = jnp.bitwise_and(wid, 3)
    g0 = wid * _CHUNK
    copies = [
        pltpu.async_copy(states_hbm.at[pl.ds(g0, _CHUNK)], statev, sem),
        pltpu.async_copy(w_hbm.at[pl.ds(g0, _CHUNK)], wv, sem),
        pltpu.async_copy(ce_hbm.at[pl.ds(win0, 8), :], cew8, sem),
        pltpu.async_copy(r_hbm.at[pl.ds(g0, _CHUNK), :], rv, sem),
    ]
    for cp in copies:
        cp.wait()
    for i in range(16):
        row = 2 * m4 + i // 8
        col = (16 * i) % 128
        cev[pl.ds(16 * i, 16)] = cew8[row, pl.ds(col, 16)]

    a_num = jnp.zeros((16,), _f32)
    a_den = jnp.zeros((16,), _f32)
    a_inv = jnp.zeros((16,), _f32)
    for i in range(16):
        plv = _iota16() + 16 * i
        states = statev[pl.ds(16 * i, 16)]
        w = wv[pl.ds(16 * i, 16)]
        cevec = cev[pl.ds(16 * i, 16)]
        a_num = a_num + cevec * w
        a_den = a_den + w
        a_inv = a_inv + plsc.load_gather(rv, [plv, states])

    i16 = _iota16()
    pv = jnp.where(i16 == 0, jnp.sum(a_num),
                   jnp.where(i16 == 1, jnp.sum(a_den),
                             jnp.where(i16 == 2, jnp.sum(a_inv), 0.0)))
    pvec[...] = pv.astype(_f32)
    pltpu.sync_copy(pvec, partials_hbm.at[pl.ds(wid * 16, 16)])
    plsc.subcore_barrier()

    @pl.when(s == 0)
    def _():
        pltpu.sync_copy(partials_hbm.at[pl.ds(c * 256, 256)], redbuf)
        acc = jnp.zeros((16,), _f32)
        for r in range(16):
            acc = acc + redbuf[pl.ds(r * 16, 16)]
        pvec[...] = acc
        pltpu.sync_copy(pvec, out_hbm.at[pl.ds(c * 16, 16)])


def _run_select(states, w, ce, r):
    mesh = plsc.VectorSubcoreMesh(core_axis_name="c", subcore_axis_name="s")
    return pl.kernel(
        _select_body,
        out_type=(
            jax.ShapeDtypeStruct((512,), _f32),
            jax.ShapeDtypeStruct((32,), _f32),
        ),
        mesh=mesh,
        compiler_params=pltpu.CompilerParams(needs_layout_passes=False),
        scratch_types=(
            pltpu.VMEM((_CHUNK, _NSTATES), _f32),
            pltpu.VMEM((_CHUNK,), _i32),
            pltpu.VMEM((_CHUNK,), _f32),
            pltpu.VMEM((_CHUNK,), _f32),
            pltpu.VMEM((8, 128), _f32),
            pltpu.VMEM((16,), _f32),
            pltpu.VMEM((256,), _f32),
            pltpu.SemaphoreType.DMA,
        ),
    )(states, w, ce, r)


def kernel(predictions, targets, inputs, transition_tensor, state_types_tensor):
    pred2 = predictions.reshape(_N, _V)
    inp2 = inputs.reshape(_N, _V)
    tgt64 = targets.reshape(_N // 128, 128).astype(_i32)
    trans = transition_tensor.astype(_i32)
    stt = state_types_tensor.astype(_i32)

    (tok,) = _run_argmax(inp2)
    rejt, packed, _isrej = _run_prep(trans, stt)
    states, w, _f = _run_scan(tok, tgt64, packed, stt)
    ce, r = _run_soft(pred2, tgt64, rejt)
    _, out2 = _run_select(states, w, ce, r)

    tot = out2.reshape(2, 16).sum(axis=0)
    wce = tot[0] / (tot[1] + 1e-6)
    inv_mean = tot[2] / float(_N)
    return _ALPHA * wce + (1.0 - _ALPHA) * (-jnp.log(1.0 - inv_mean + 1e-6))


# speedup vs baseline: 1.0653x; 1.0653x over previous
import jax
import jax.numpy as jnp
from jax import lax
from jax.experimental import pallas as pl
from jax.experimental.pallas import tpu as pltpu, tpu_sc as plsc

_ALPHA = 0.7
_NSTATES = 128
_B, _S, _V = 4, 2048, 1024
_N = _B * _S
_CHUNK = _N // 32
_ROWS = 1024
_PACKED_WORDS = _NSTATES * _V // 4

_i32 = jnp.int32
_f32 = jnp.float32


def _iota16():
    return lax.iota(_i32, 16)


def _prep_body(transt_hbm, st_hbm, rejt_hbm, packed_hbm, isrej_hbm,
               tcolt, stv, isrejv, rejtv, packedv, sem):
    c = lax.axis_index("c")
    s = lax.axis_index("s")
    wid = c * 16 + s
    v0 = wid * 32

    cp1 = pltpu.async_copy(st_hbm, stv, sem)
    cp2 = pltpu.async_copy(transt_hbm.at[pl.ds(v0, 32), :], tcolt, sem)
    cp1.wait()
    cp2.wait()

    for k in range(8):
        sv = stv[pl.ds(k * 16, 16)]
        isrejv[pl.ds(k * 16, 16)] = jnp.where(sv == -1, 1.0, 0.0).astype(_f32)

    @pl.when(wid == 0)
    def _():
        pltpu.sync_copy(isrejv, isrej_hbm)

    for vl in range(32):
        for k in range(8):
            tv = tcolt[vl, pl.ds(16 * k, 16)]
            rj = plsc.load_gather(isrejv, [tv])
            rejtv[vl, pl.ds(16 * k, 16)] = rj
        for half in range(2):
            col4 = lax.shift_left(_iota16() + 16 * half, 2)
            g0 = plsc.load_gather(tcolt, [jnp.full((16,), vl, _i32), col4])
            g1 = plsc.load_gather(tcolt, [jnp.full((16,), vl, _i32), col4 + 1])
            g2 = plsc.load_gather(tcolt, [jnp.full((16,), vl, _i32), col4 + 2])
            g3 = plsc.load_gather(tcolt, [jnp.full((16,), vl, _i32), col4 + 3])
            word = jnp.bitwise_or(
                jnp.bitwise_or(g0, lax.shift_left(g1, 8)),
                jnp.bitwise_or(lax.shift_left(g2, 16), lax.shift_left(g3, 24)))
            packedv[pl.ds(32 * vl + 16 * half, 16)] = word
    cp3 = pltpu.async_copy(rejtv, rejt_hbm.at[pl.ds(v0, 32), :], sem)
    cp4 = pltpu.async_copy(packedv, packed_hbm.at[pl.ds(wid * 1024, 1024)], sem)
    cp3.wait()
    cp4.wait()


def _run_prep(transition_t, state_types):
    mesh = plsc.VectorSubcoreMesh(core_axis_name="c", subcore_axis_name="s")
    return pl.kernel(
        _prep_body,
        out_type=(
            jax.ShapeDtypeStruct((_V, _NSTATES), _f32),
            jax.ShapeDtypeStruct((_PACKED_WORDS,), _i32),
            jax.ShapeDtypeStruct((_NSTATES,), _f32),
        ),
        mesh=mesh,
        compiler_params=pltpu.CompilerParams(needs_layout_passes=False),
        scratch_types=(
            pltpu.VMEM((32, _NSTATES), _i32),
            pltpu.VMEM((_NSTATES,), _i32),
            pltpu.VMEM((_NSTATES,), _f32),
            pltpu.VMEM((32, _NSTATES), _f32),
            pltpu.VMEM((1024,), _i32),
            pltpu.SemaphoreType.DMA,
        ),
    )(transition_t, state_types)


def _argmax_body(inp_ref, tok_ref):
    iota3 = lax.broadcasted_iota(_i32, (8, 128, _V), 2)
    x3 = inp_ref[...].reshape(8, 128, _V)
    xm = jnp.max(x3, axis=2, keepdims=True)
    tok3 = jnp.min(jnp.where(x3 == xm, iota3, _V), axis=2, keepdims=True)
    tok_ref[...] = tok3.reshape(8, 128)


def _run_argmax(inp2):
    return pl.pallas_call(
        _argmax_body,
        grid=(_N // _ROWS,),
        compiler_params=pltpu.CompilerParams(
            dimension_semantics=("arbitrary",)),
        in_specs=[pl.BlockSpec((_ROWS, _V), lambda i: (i, 0))],
        out_specs=[pl.BlockSpec((8, 128), lambda i: (i, 0))],
        out_shape=[jax.ShapeDtypeStruct((_N // 128, 128), _i32)],
    )(inp2)


def _soft_body(pred_ref, tgt_ref, rejt_ref, ce_ref, r_ref):
    p3 = pred_ref[...].reshape(8, 128, _V)
    iota3 = lax.broadcasted_iota(_i32, (8, 128, _V), 2)
    m = jnp.max(p3, axis=2, keepdims=True)
    e3 = jnp.exp(p3 - m)
    z = jnp.sum(e3, axis=2, keepdims=True)
    tgt3 = tgt_ref[...][..., None]
    pt = jnp.sum(jnp.where(iota3 == tgt3, p3, 0.0), axis=2, keepdims=True)
    ce_ref[...] = (m + jnp.log(z) - pt).reshape(8, 128)

    pn = (e3 / z).reshape(_ROWS, _V)
    r_ref[...] = jnp.dot(pn, rejt_ref[...], preferred_element_type=_f32)


def _run_soft(pred2, tgt64, rejt):
    return pl.pallas_call(
        _soft_body,
        grid=(_N // _ROWS,),
        compiler_params=pltpu.CompilerParams(
            dimension_semantics=("arbitrary",)),
        in_specs=[
            pl.BlockSpec((_ROWS, _V), lambda i: (i, 0)),
            pl.BlockSpec((8, 128), lambda i: (i, 0)),
            pl.BlockSpec((_V, _NSTATES), lambda i: (0, 0)),
        ],
        out_specs=[
            pl.BlockSpec((8, 128), lambda i: (i, 0)),
            pl.BlockSpec((_ROWS, _NSTATES), lambda i: (i, 0)),
        ],
        out_shape=[
            jax.ShapeDtypeStruct((_N // 128, 128), _f32),
            jax.ShapeDtypeStruct((_N, _NSTATES), _f32),
        ],
    )(pred2, tgt64, rejt)


def _scan_body(tok_hbm, tgt_hbm, packed_hbm, st_hbm,
               states_hbm, w_hbm, fstage_hbm,
               packedv, stored, tokv, tgtv, stv, isrejv,
               fbuf, frow, statev, wv, tokw8, tgtw8, sem):
    c = lax.axis_index("c")
    s = lax.axis_index("s")
    wid = c * 16 + s
    chunk = jnp.bitwise_and(s, 7)
    win0 = 8 * lax.div(wid, 4)
    m4 = jnp.bitwise_and(wid, 3)
    copies = [
        pltpu.async_copy(packed_hbm, packedv, sem),
        pltpu.async_copy(tok_hbm.at[pl.ds(win0, 8), :], tokw8, sem),
        pltpu.async_copy(tgt_hbm.at[pl.ds(win0, 8), :], tgtw8, sem),
        pltpu.async_copy(st_hbm, stv, sem),
    ]
    for cp in copies:
        cp.wait()
    for i in range(16):
        row = 2 * m4 + i // 8
        col = (16 * i) % 128
        tokv[pl.ds(16 * i, 16)] = tokw8[row, pl.ds(col, 16)]
        tgtv[pl.ds(16 * i, 16)] = tgtw8[row, pl.ds(col, 16)]
    for k in range(8):
        sv = stv[pl.ds(k * 16, 16)]
        isrejv[pl.ds(k * 16, 16)] = jnp.where(sv == -1, 1.0, 0.0).astype(_f32)

    def step(p, carry):
        pvec = carry[0]
        fs = list(carry[1:])
        new = [pvec + 1]
        for sub in range(4):
            tokbc = plsc.load_gather(tokv, [pvec + 64 * sub])
            tokw = lax.shift_left(tokbc, 5)
            for k in range(8):
                f = fs[sub * 8 + k]
                idx = tokw + lax.shift_right_logical(f, 2)
                g = plsc.load_gather(packedv, [idx])
                fsh = lax.shift_left(jnp.bitwise_and(f, 3), 3)
                nxt = jnp.bitwise_and(lax.shift_right_logical(g, fsh), 255)
                stored[64 * sub + p, pl.ds(16 * k, 16)] = nxt
                new.append(nxt)
        return tuple(new)

    finit = (jnp.zeros((16,), _i32),) + tuple(
        _iota16() + 16 * k for k in range(8)) * 4
    lax.fori_loop(0, 64, step, finit)

    fk = [stored[63, pl.ds(16 * k, 16)] for k in range(8)]
    for level in range(1, 4):
        rowv = jnp.full((16,), 64 * level + 63, _i32)
        fk = [plsc.load_gather(stored, [rowv, f]) for f in fk]
    for k in range(8):
        frow[pl.ds(16 * k, 16)] = fk[k]
    pltpu.sync_copy(frow, fstage_hbm.at[pl.ds(wid * _NSTATES, _NSTATES)])
    plsc.subcore_barrier()

    rowbase = c * 16 + jnp.bitwise_and(s, 8)
    pltpu.sync_copy(
        fstage_hbm.at[pl.ds(rowbase * _NSTATES, 8 * _NSTATES)], fbuf)
    s0 = jnp.asarray(0, _i32)
    for k in range(7):
        val = jnp.max(
            plsc.load_gather(fbuf, [jnp.full((16,), k * _NSTATES + s0, _i32)]))
        s0 = jnp.where(chunk > k, val, s0)

    s_sub = [s0]
    for t in range(1, 4):
        prev = s_sub[t - 1]
        v = jnp.max(plsc.load_gather(
            stored, [jnp.full((16,), 64 * t - 1, _i32),
                     jnp.full((16,), prev, _i32)]))
        s_sub.append(v)
    s_subv = [jnp.full((16,), x, _i32) for x in s_sub]

    for i in range(16):
        sub = i // 4
        s0vec = s_subv[sub]
        plv = _iota16() + 16 * i
        pm1 = jnp.maximum(plv - 1, 0)
        prev = plsc.load_gather(stored, [pm1, s0vec])
        states = jnp.where(plv == 64 * sub, s0vec, prev)
        tgt = tgtv[pl.ds(16 * i, 16)]
        idx = lax.shift_left(tgt, 5) + lax.shift_right_logical(states, 2)
        g = plsc.load_gather(packedv, [idx])
        sh = lax.shift_left(jnp.bitwise_and(states, 3), 3)
        nxt = jnp.bitwise_and(lax.shift_right_logical(g, sh), 255)
        rj = plsc.load_gather(isrejv, [nxt])
        statev[pl.ds(16 * i, 16)] = states
        wv[pl.ds(16 * i, 16)] = 1.0 - 0.95 * rj

    g0 = wid * _CHUNK
    cp5 = pltpu.async_copy(statev, states_hbm.at[pl.ds(g0, _CHUNK)], sem)
    cp6 = pltpu.async_copy(wv, w_hbm.at[pl.ds(g0, _CHUNK)], sem)
    cp5.wait()
    cp6.wait()


def _run_scan(tokens, targets, packed, state_types):
    mesh = plsc.VectorSubcoreMesh(core_axis_name="c", subcore_axis_name="s")
    return pl.kernel(
        _scan_body,
        out_type=(
            jax.ShapeDtypeStruct((_N,), _i32),
            jax.ShapeDtypeStruct((_N,), _f32),
            jax.ShapeDtypeStruct((32 * _NSTATES,), _i32),
        ),
        mesh=mesh,
        compiler_params=pltpu.CompilerParams(needs_layout_passes=False),
        scratch_types=(
            pltpu.VMEM((_PACKED_WORDS,), _i32),
            pltpu.VMEM((_CHUNK, _NSTATES), _i32),
            pltpu.VMEM((_CHUNK,), _i32),
            pltpu.VMEM((_CHUNK,), _i32),
            pltpu.VMEM((_NSTATES,), _i32),
            pltpu.VMEM((_NSTATES,), _f32),
            pltpu.VMEM((8 * _NSTATES,), _i32),
            pltpu.VMEM((_NSTATES,), _i32),
            pltpu.VMEM((_CHUNK,), _i32),
            pltpu.VMEM((_CHUNK,), _f32),
            pltpu.VMEM((8, 128), _i32),
            pltpu.VMEM((8, 128), _i32),
            pltpu.SemaphoreType.DMA,
        ),
    )(tokens, targets, packed, state_types)


def _select_body(states_hbm, w_hbm, ce_hbm, r_hbm,
                 partials_hbm, out_hbm,
                 rv, statev, wv, cev, cew8, pvec, redbuf, sem):
    c = lax.axis_index("c")
    s = lax.axis_index("s")
    wid = c * 16 + s
    win0 = 8 * lax.div(wid, 4)
    m4 = jnp.bitwise_and(wid, 3)
    g0 = wid * _CHUNK
    copies = [
        pltpu.async_copy(states_hbm.at[pl.ds(g0, _CHUNK)], statev, sem),
        pltpu.async_copy(w_hbm.at[pl.ds(g0, _CHUNK)], wv, sem),
        pltpu.async_copy(ce_hbm.at[pl.ds(win0, 8), :], cew8, sem),
        pltpu.async_copy(r_hbm.at[pl.ds(g0, _CHUNK), :], rv, sem),
    ]
    for cp in copies:
        cp.wait()
    for i in range(16):
        row = 2 * m4 + i // 8
        col = (16 * i) % 128
        cev[pl.ds(16 * i, 16)] = cew8[row, pl.ds(col, 16)]

    a_num = jnp.zeros((16,), _f32)
    a_den = jnp.zeros((16,), _f32)
    a_inv = jnp.zeros((16,), _f32)
    for i in range(16):
        plv = _iota16() + 16 * i
        states = statev[pl.ds(16 * i, 16)]
        w = wv[pl.ds(16 * i, 16)]
        cevec = cev[pl.ds(16 * i, 16)]
        a_num = a_num + cevec * w
        a_den = a_den + w
        a_inv = a_inv + plsc.load_gather(rv, [plv, states])

    i16 = _iota16()
    pv = jnp.where(i16 == 0, jnp.sum(a_num),
                   jnp.where(i16 == 1, jnp.sum(a_den),
                             jnp.where(i16 == 2, jnp.sum(a_inv), 0.0)))
    pvec[...] = pv.astype(_f32)
    pltpu.sync_copy(pvec, partials_hbm.at[pl.ds(wid * 16, 16)])
    plsc.subcore_barrier()

    @pl.when(s == 0)
    def _():
        pltpu.sync_copy(partials_hbm.at[pl.ds(c * 256, 256)], redbuf)
        acc = jnp.zeros((16,), _f32)
        for r in range(16):
            acc = acc + redbuf[pl.ds(r * 16, 16)]
        pvec[...] = acc
        pltpu.sync_copy(pvec, out_hbm.at[pl.ds(c * 16, 16)])


def _run_select(states, w, ce, r):
    mesh = plsc.VectorSubcoreMesh(core_axis_name="c", subcore_axis_name="s")
    return pl.kernel(
        _select_body,
        out_type=(
            jax.ShapeDtypeStruct((512,), _f32),
            jax.ShapeDtypeStruct((32,), _f32),
        ),
        mesh=mesh,
        compiler_params=pltpu.CompilerParams(needs_layout_passes=False),
        scratch_types=(
            pltpu.VMEM((_CHUNK, _NSTATES), _f32),
            pltpu.VMEM((_CHUNK,), _i32),
            pltpu.VMEM((_CHUNK,), _f32),
            pltpu.VMEM((_CHUNK,), _f32),
            pltpu.VMEM((8, 128), _f32),
            pltpu.VMEM((16,), _f32),
            pltpu.VMEM((256,), _f32),
            pltpu.SemaphoreType.DMA,
        ),
    )(states, w, ce, r)


def kernel(predictions, targets, inputs, transition_tensor, state_types_tensor):
    pred2 = predictions.reshape(_N, _V)
    inp2 = inputs.reshape(_N, _V)
    tgt64 = targets.reshape(_N // 128, 128).astype(_i32)
    trans = transition_tensor.astype(_i32)
    stt = state_types_tensor.astype(_i32)

    (tok,) = _run_argmax(inp2)
    rejt, packed, _isrej = _run_prep(trans.T, stt)
    states, w, _f = _run_scan(tok, tgt64, packed, stt)
    ce, r = _run_soft(pred2, tgt64, rejt)
    _, out2 = _run_select(states, w, ce, r)

    tot = out2.reshape(2, 16).sum(axis=0)
    wce = tot[0] / (tot[1] + 1e-6)
    inv_mean = tot[2] / float(_N)
    return _ALPHA * wce + (1.0 - _ALPHA) * (-jnp.log(1.0 - inv_mean + 1e-6))
